# baseline (speedup 1.0000x reference)
import functools

import jax
import jax.numpy as jnp
from jax import lax
from jax.experimental import pallas as pl
from jax.experimental.pallas import tpu as pltpu
from jax.experimental.pallas import tpu_sc as plsc

NUM_FIELDS = 26
EMBEDDING_DIM = 32
PAD_DIM = 33
LANES = 16
ROW_W = 128
DMAS_PER_CHUNK = 8
TILE_R = 128
BLK = TILE_R * EMBEDDING_DIM
PBLK = TILE_R * PAD_DIM


def _make_sc_repack(n_rows):
    info = plsc.get_sparse_core_info()
    nc, ns = info.num_cores, info.num_subcores
    nw = nc * ns
    n_full = n_rows // TILE_R
    npairs = n_full // 2
    per_w = -(-npairs // nw)
    last_w = npairs - per_w * (nw - 1)
    n_rows_pad = -(-n_rows // TILE_R) * TILE_R
    out_words = n_rows_pad * EMBEDDING_DIM
    tail_rows = n_rows - n_full * TILE_R
    tail_words = tail_rows * EMBEDDING_DIM
    NBUF = 3
    mesh = plsc.VectorSubcoreMesh(core_axis_name="c", subcore_axis_name="s")

    @functools.partial(
        pl.kernel,
        mesh=mesh,
        compiler_params=pltpu.CompilerParams(
            use_tc_tiling_on_sc=True, needs_layout_passes=False),
        out_type=jax.ShapeDtypeStruct((out_words,), jnp.float32),
        scratch_types=(
            [pltpu.VMEM((NBUF, 2, EMBEDDING_DIM, TILE_R), jnp.float32)]
            + [pltpu.VMEM((2 * PBLK,), jnp.float32)]
            + [pltpu.VMEM((2 * BLK,), jnp.float32) for _ in range(NBUF)]
            + [pltpu.VMEM((tail_words,), jnp.float32),
               pltpu.VMEM((LANES,), jnp.int32),
               pltpu.SemaphoreType.DMA,
               pltpu.SemaphoreType.DMA]
        ),
    )
    def sc_repack(table_t, tail_hbm, iota_hbm, out_hbm, stag, pbuf,
                  outl0, outl1, outl2, tstag, iov, lsem, ssem):
        outls = [outl0, outl1, outl2]
        wid = lax.axis_index("s") * nc + lax.axis_index("c")
        j0 = wid * per_w * 2
        nt = jnp.where(wid == nw - 1, last_w, per_w)
        n_outer = -(-per_w // NBUF)
        pltpu.sync_copy(iota_hbm, iov)
        riota = iov[pl.ds(0, LANES)]
        scat_base = [
            (lax.iota(jnp.int32, LANES) + l0) * PAD_DIM
            for l0 in range(0, TILE_R, LANES)
        ]

        def load(t, buf, c):
            return pltpu.async_copy(
                table_t.at[pl.ds(0, EMBEDDING_DIM),
                           pl.ds((j0 + 2 * t + c) * TILE_R, TILE_R)],
                stag.at[buf, c], lsem)

        for p in range(NBUF - 1):
            @pl.when(p < nt)
            def _(p=p):
                load(p, p, 0)
                load(p, p, 1)

        def outer(g, carry):
            for b in range(NBUF):
                t = g * NBUF + b

                @pl.when(t < nt)
                def _(t=t, b=b):
                    for c in range(2):
                        pltpu.make_async_copy(
                            table_t.at[pl.ds(0, EMBEDDING_DIM), pl.ds(0, TILE_R)],
                            stag.at[b, c], lsem).wait()

                    @pl.when(t + NBUF - 1 < nt)
                    def _():
                        load(t + NBUF - 1, (b + NBUF - 1) % NBUF, 0)
                        load(t + NBUF - 1, (b + NBUF - 1) % NBUF, 1)

                    @pl.when(t >= NBUF - 1)
                    def _():
                        pltpu.make_async_copy(
                            outls[0], out_hbm.at[pl.ds(0, 2 * BLK)],
                            ssem).wait()

                    for c in range(2):
                        sbuf = stag.at[b, c]
                        for d in range(EMBEDDING_DIM):
                            vs = [sbuf[d, pl.ds(l0, LANES)]
                                  for l0 in range(0, TILE_R, LANES)]
                            for m in range(TILE_R // LANES):
                                plsc.store_scatter(
                                    pbuf, [scat_base[m] + (c * PBLK + d)], vs[m])
                    ol = outls[b]
                    for c in range(2):
                        for row in range(TILE_R):
                            for d0 in range(0, EMBEDDING_DIM, LANES):
                                v = plsc.load_gather(
                                    pbuf,
                                    [riota + (c * PBLK + row * PAD_DIM + d0)])
                                ol[pl.ds(c * BLK + row * EMBEDDING_DIM + d0,
                                         LANES)] = v
                    pltpu.async_copy(
                        ol, out_hbm.at[pl.ds((j0 + 2 * t) * BLK, 2 * BLK)],
                        ssem)
            return carry

        lax.fori_loop(0, n_outer, outer, 0)
        for p in range(NBUF - 1):
            pltpu.make_async_copy(
                outls[0], out_hbm.at[pl.ds(0, 2 * BLK)], ssem).wait()

        @pl.when(wid == 0)
        def _():
            pltpu.sync_copy(tail_hbm, tstag)
            pltpu.sync_copy(tstag, out_hbm.at[pl.ds(n_full * BLK, tail_words)])

    return sc_repack


def _make_sc_gather(n_flat):
    info = plsc.get_sparse_core_info()
    nc, ns = info.num_cores, info.num_subcores
    nw = nc * ns
    per_w = n_flat // nw
    assert per_w * nw == n_flat and per_w % (ROW_W * DMAS_PER_CHUNK) == 0
    n_idx_rows = per_w // ROW_W
    chunk = ROW_W * DMAS_PER_CHUNK
    n_chunks = per_w // chunk

    mesh = plsc.VectorSubcoreMesh(core_axis_name="c", subcore_axis_name="s")

    @functools.partial(
        pl.kernel,
        mesh=mesh,
        compiler_params=pltpu.CompilerParams(use_tc_tiling_on_sc=False),
        out_type=jax.ShapeDtypeStruct((n_flat, EMBEDDING_DIM), jnp.float32),
        scratch_types=[
            pltpu.VMEM((n_idx_rows, ROW_W), jnp.int32),
            pltpu.VMEM((n_idx_rows, ROW_W), jnp.int32),
            pltpu.VMEM((chunk, EMBEDDING_DIM), jnp.float32),
            pltpu.SemaphoreType.DMA,
        ],
    )
    def sc_gather(idx_hbm, off_hbm, table_hbm, out_hbm, idx_v, off_v, rows_v,
                  gsem):
        wid = lax.axis_index("s") * nc + lax.axis_index("c")
        base_row = wid * n_idx_rows
        pltpu.sync_copy(idx_hbm.at[pl.ds(base_row, n_idx_rows)], idx_v)
        pltpu.sync_copy(off_hbm, off_v)

        def add_body(r, carry):
            for k in range(ROW_W // LANES):
                sl = pl.ds(k * LANES, LANES)
                idx_v[r, sl] = idx_v[r, sl] + off_v[r, sl]
            return carry

        lax.fori_loop(0, n_idx_rows, add_body, 0)

        def chunk_body(t, carry):
            copies = []
            for b in range(DMAS_PER_CHUNK):
                copies.append(pltpu.async_copy(
                    table_hbm.at[idx_v.at[t * DMAS_PER_CHUNK + b]],
                    rows_v.at[pl.ds(b * ROW_W, ROW_W)],
                    gsem,
                ))
            for c in copies:
                c.wait()
            pltpu.sync_copy(
                rows_v,
                out_hbm.at[pl.ds((base_row + t * DMAS_PER_CHUNK) * ROW_W, chunk)],
            )
            return carry

        lax.fori_loop(0, n_chunks, chunk_body, 0)

    return sc_gather


def kernel(categorical_inputs, embedding_weight, offsets):
    batch, n_fields = categorical_inputs.shape
    n_flat = batch * n_fields
    idx_flat = categorical_inputs.astype(jnp.int32).reshape(n_flat // ROW_W, ROW_W)
    info = plsc.get_sparse_core_info()
    per_w = n_flat // (info.num_cores * info.num_subcores)
    off_pattern = jnp.tile(
        offsets[:n_fields].astype(jnp.int32), per_w // n_fields
    ).reshape(per_w // ROW_W, ROW_W)

    n_rows = embedding_weight.shape[0]
    n_full = n_rows // TILE_R
    tail = jax.lax.slice(
        embedding_weight, (n_full * TILE_R, 0), (n_rows, EMBEDDING_DIM)
    ).reshape(-1)
    iota16 = jnp.arange(LANES, dtype=jnp.int32)
    sc_repack = _make_sc_repack(n_rows)
    scratch = sc_repack(embedding_weight.T, tail, iota16)
    table_lin = scratch.reshape(-1, EMBEDDING_DIM)

    sc_gather = _make_sc_gather(n_flat)
    out_flat = sc_gather(idx_flat, off_pattern, table_lin)
    return out_flat.reshape(batch, n_fields, EMBEDDING_DIM)


# speedup vs baseline: 1.1355x; 1.1355x over previous
import functools

import jax
import jax.numpy as jnp
from jax import lax
from jax.experimental import pallas as pl
from jax.experimental.pallas import tpu as pltpu
from jax.experimental.pallas import tpu_sc as plsc

NUM_FIELDS = 26
EMBEDDING_DIM = 32
PAD_DIM = 33
LANES = 16
ROW_W = 128
DMAS_PER_CHUNK = 8
TILE_R = 128
BLK = TILE_R * EMBEDDING_DIM
PBLK = TILE_R * PAD_DIM


def _make_sc_repack(n_rows):
    info = plsc.get_sparse_core_info()
    nc, ns = info.num_cores, info.num_subcores
    nw = nc * ns
    n_full = n_rows // TILE_R
    npairs = n_full // 2
    per_w = -(-npairs // nw)
    last_w = npairs - per_w * (nw - 1)
    n_rows_pad = -(-n_rows // TILE_R) * TILE_R
    out_words = n_rows_pad * EMBEDDING_DIM
    tail_rows = n_rows - n_full * TILE_R
    tail_words = tail_rows * EMBEDDING_DIM
    NBUF = 3
    mesh = plsc.VectorSubcoreMesh(core_axis_name="c", subcore_axis_name="s")

    @functools.partial(
        pl.kernel,
        mesh=mesh,
        compiler_params=pltpu.CompilerParams(
            use_tc_tiling_on_sc=True, needs_layout_passes=False),
        out_type=jax.ShapeDtypeStruct((out_words,), jnp.float32),
        scratch_types=(
            [pltpu.VMEM((NBUF, 2, EMBEDDING_DIM, TILE_R), jnp.float32)]
            + [pltpu.VMEM((2 * PBLK,), jnp.float32)]
            + [pltpu.VMEM((2 * BLK,), jnp.float32) for _ in range(NBUF)]
            + [pltpu.VMEM((tail_words,), jnp.float32),
               pltpu.VMEM((LANES,), jnp.int32),
               pltpu.SemaphoreType.DMA,
               pltpu.SemaphoreType.DMA]
        ),
    )
    def sc_repack(table_t, tail_hbm, iota_hbm, out_hbm, stag, pbuf,
                  outl0, outl1, outl2, tstag, iov, lsem, ssem):
        outls = [outl0, outl1, outl2]
        wid = lax.axis_index("s") * nc + lax.axis_index("c")
        j0 = wid * per_w * 2
        nt = jnp.where(wid == nw - 1, last_w, per_w)
        n_outer = -(-per_w // NBUF)
        pltpu.sync_copy(iota_hbm, iov)
        scat_base = [
            (lax.iota(jnp.int32, LANES) + l0) * PAD_DIM
            for l0 in range(0, TILE_R, LANES)
        ]

        def load(t, buf, c):
            return pltpu.async_copy(
                table_t.at[pl.ds(0, EMBEDDING_DIM),
                           pl.ds((j0 + 2 * t + c) * TILE_R, TILE_R)],
                stag.at[buf, c], lsem)

        for p in range(NBUF - 1):
            @pl.when(p < nt)
            def _(p=p):
                load(p, p, 0)
                load(p, p, 1)

        def outer(g, carry):
            for b in range(NBUF):
                t = g * NBUF + b

                @pl.when(t < nt)
                def _(t=t, b=b):
                    for c in range(2):
                        pltpu.make_async_copy(
                            table_t.at[pl.ds(0, EMBEDDING_DIM), pl.ds(0, TILE_R)],
                            stag.at[b, c], lsem).wait()

                    @pl.when(t + NBUF - 1 < nt)
                    def _():
                        load(t + NBUF - 1, (b + NBUF - 1) % NBUF, 0)
                        load(t + NBUF - 1, (b + NBUF - 1) % NBUF, 1)

                    @pl.when(t >= NBUF - 1)
                    def _():
                        pltpu.make_async_copy(
                            outls[0], out_hbm.at[pl.ds(0, 2 * BLK)],
                            ssem).wait()

                    for c in range(2):
                        sbuf = stag.at[b, c]
                        for d in range(EMBEDDING_DIM):
                            vs = [sbuf[d, pl.ds(l0, LANES)]
                                  for l0 in range(0, TILE_R, LANES)]
                            for m in range(TILE_R // LANES):
                                plsc.store_scatter(
                                    pbuf, [scat_base[m] + (c * PBLK + d)], vs[m])
                    riota = iov[pl.ds(0, LANES)]
                    ol = outls[b]
                    for c in range(2):
                        for row in range(TILE_R):
                            for d0 in range(0, EMBEDDING_DIM, LANES):
                                v = plsc.load_gather(
                                    pbuf,
                                    [riota + (c * PBLK + row * PAD_DIM + d0)])
                                ol[pl.ds(c * BLK + row * EMBEDDING_DIM + d0,
                                         LANES)] = v
                    pltpu.async_copy(
                        ol, out_hbm.at[pl.ds((j0 + 2 * t) * BLK, 2 * BLK)],
                        ssem)
            return carry

        lax.fori_loop(0, n_outer, outer, 0)
        for p in range(NBUF - 1):
            pltpu.make_async_copy(
                outls[0], out_hbm.at[pl.ds(0, 2 * BLK)], ssem).wait()

        @pl.when(wid == 0)
        def _():
            pltpu.sync_copy(tail_hbm, tstag)
            pltpu.sync_copy(tstag, out_hbm.at[pl.ds(n_full * BLK, tail_words)])

    return sc_repack


def _make_sc_gather(n_flat):
    info = plsc.get_sparse_core_info()
    nc, ns = info.num_cores, info.num_subcores
    nw = nc * ns
    per_w = n_flat // nw
    assert per_w * nw == n_flat and per_w % (ROW_W * DMAS_PER_CHUNK) == 0
    n_idx_rows = per_w // ROW_W
    chunk = ROW_W * DMAS_PER_CHUNK
    n_chunks = per_w // chunk

    mesh = plsc.VectorSubcoreMesh(core_axis_name="c", subcore_axis_name="s")

    @functools.partial(
        pl.kernel,
        mesh=mesh,
        compiler_params=pltpu.CompilerParams(use_tc_tiling_on_sc=False),
        out_type=jax.ShapeDtypeStruct((n_flat, EMBEDDING_DIM), jnp.float32),
        scratch_types=[
            pltpu.VMEM((n_idx_rows, ROW_W), jnp.int32),
            pltpu.VMEM((n_idx_rows, ROW_W), jnp.int32),
            pltpu.VMEM((chunk, EMBEDDING_DIM), jnp.float32),
            pltpu.SemaphoreType.DMA,
        ],
    )
    def sc_gather(idx_hbm, off_hbm, table_hbm, out_hbm, idx_v, off_v, rows_v,
                  gsem):
        wid = lax.axis_index("s") * nc + lax.axis_index("c")
        base_row = wid * n_idx_rows
        pltpu.sync_copy(idx_hbm.at[pl.ds(base_row, n_idx_rows)], idx_v)
        pltpu.sync_copy(off_hbm, off_v)

        def add_body(r, carry):
            for k in range(ROW_W // LANES):
                sl = pl.ds(k * LANES, LANES)
                idx_v[r, sl] = idx_v[r, sl] + off_v[r, sl]
            return carry

        lax.fori_loop(0, n_idx_rows, add_body, 0)

        def chunk_body(t, carry):
            copies = []
            for b in range(DMAS_PER_CHUNK):
                copies.append(pltpu.async_copy(
                    table_hbm.at[idx_v.at[t * DMAS_PER_CHUNK + b]],
                    rows_v.at[pl.ds(b * ROW_W, ROW_W)],
                    gsem,
                ))
            for c in copies:
                c.wait()
            pltpu.sync_copy(
                rows_v,
                out_hbm.at[pl.ds((base_row + t * DMAS_PER_CHUNK) * ROW_W, chunk)],
            )
            return carry

        lax.fori_loop(0, n_chunks, chunk_body, 0)

    return sc_gather


def kernel(categorical_inputs, embedding_weight, offsets):
    batch, n_fields = categorical_inputs.shape
    n_flat = batch * n_fields
    idx_flat = categorical_inputs.astype(jnp.int32).reshape(n_flat // ROW_W, ROW_W)
    info = plsc.get_sparse_core_info()
    per_w = n_flat // (info.num_cores * info.num_subcores)
    off_pattern = jnp.tile(
        offsets[:n_fields].astype(jnp.int32), per_w // n_fields
    ).reshape(per_w // ROW_W, ROW_W)

    n_rows = embedding_weight.shape[0]
    n_full = n_rows // TILE_R
    tail = jax.lax.slice(
        embedding_weight, (n_full * TILE_R, 0), (n_rows, EMBEDDING_DIM)
    ).reshape(-1)
    iota16 = jnp.arange(LANES, dtype=jnp.int32)
    sc_repack = _make_sc_repack(n_rows)
    scratch = sc_repack(embedding_weight.T, tail, iota16)
    table_lin = scratch.reshape(-1, EMBEDDING_DIM)

    sc_gather = _make_sc_gather(n_flat)
    out_flat = sc_gather(idx_flat, off_pattern, table_lin)
    return out_flat.reshape(batch, n_fields, EMBEDDING_DIM)


# speedup vs baseline: 1.4053x; 1.2376x over previous
import functools

import jax
import jax.numpy as jnp
from jax import lax
from jax.experimental import pallas as pl
from jax.experimental.pallas import tpu as pltpu
from jax.experimental.pallas import tpu_sc as plsc

NUM_FIELDS = 26
EMBEDDING_DIM = 32
PAD_DIM = 33
LANES = 16
ROW_W = 128
DMAS_PER_CHUNK = 8
TILE_R = 128
BLK = TILE_R * EMBEDDING_DIM
PBLK = TILE_R * PAD_DIM


def _make_sc_repack(n_rows):
    info = plsc.get_sparse_core_info()
    nc, ns = info.num_cores, info.num_subcores
    nw = nc * ns
    n_full = n_rows // TILE_R
    npairs = n_full // 2
    per_w = -(-npairs // nw)
    last_w = npairs - per_w * (nw - 1)
    n_rows_pad = -(-n_rows // TILE_R) * TILE_R
    out_words = n_rows_pad * EMBEDDING_DIM
    tail_rows = n_rows - n_full * TILE_R
    tail_words = tail_rows * EMBEDDING_DIM
    NBUF = 3
    mesh = plsc.VectorSubcoreMesh(core_axis_name="c", subcore_axis_name="s")

    @functools.partial(
        pl.kernel,
        mesh=mesh,
        compiler_params=pltpu.CompilerParams(
            use_tc_tiling_on_sc=True, needs_layout_passes=False),
        out_type=jax.ShapeDtypeStruct((out_words,), jnp.float32),
        scratch_types=(
            [pltpu.VMEM((NBUF, 2, EMBEDDING_DIM, TILE_R), jnp.float32)]
            + [pltpu.VMEM((2 * PBLK,), jnp.float32)]
            + [pltpu.VMEM((2 * BLK,), jnp.float32) for _ in range(NBUF)]
            + [pltpu.VMEM((tail_words,), jnp.float32),
               pltpu.VMEM((LANES,), jnp.int32),
               pltpu.SemaphoreType.DMA,
               pltpu.SemaphoreType.DMA]
        ),
    )
    def sc_repack(table_t, tail_hbm, iota_hbm, out_hbm, stag, pbuf,
                  outl0, outl1, outl2, tstag, iov, lsem, ssem):
        outls = [outl0, outl1, outl2]
        wid = lax.axis_index("s") * nc + lax.axis_index("c")
        j0 = wid * per_w * 2
        nt = jnp.where(wid == nw - 1, last_w, per_w)
        n_outer = -(-per_w // NBUF)
        pltpu.sync_copy(iota_hbm, iov)
        scat_base = [
            (lax.iota(jnp.int32, LANES) + l0) * PAD_DIM
            for l0 in range(0, TILE_R, LANES)
        ]

        def load(t, buf, c):
            return pltpu.async_copy(
                table_t.at[pl.ds(0, EMBEDDING_DIM),
                           pl.ds((j0 + 2 * t + c) * TILE_R, TILE_R)],
                stag.at[buf, c], lsem)

        for p in range(NBUF - 1):
            @pl.when(p < nt)
            def _(p=p):
                load(p, p, 0)
                load(p, p, 1)

        def outer(g, carry):
            for b in range(NBUF):
                t = g * NBUF + b

                @pl.when(t < nt)
                def _(t=t, b=b):
                    for c in range(2):
                        pltpu.make_async_copy(
                            table_t.at[pl.ds(0, EMBEDDING_DIM), pl.ds(0, TILE_R)],
                            stag.at[b, c], lsem).wait()

                    @pl.when(t + NBUF - 1 < nt)
                    def _():
                        load(t + NBUF - 1, (b + NBUF - 1) % NBUF, 0)
                        load(t + NBUF - 1, (b + NBUF - 1) % NBUF, 1)

                    @pl.when(t >= NBUF - 1)
                    def _():
                        pltpu.make_async_copy(
                            outls[0], out_hbm.at[pl.ds(0, 2 * BLK)],
                            ssem).wait()

                    for c in range(2):
                        sbuf = stag.at[b, c]
                        for d in range(EMBEDDING_DIM):
                            vs = [sbuf[d, pl.ds(l0, LANES)]
                                  for l0 in range(0, TILE_R, LANES)]
                            for m in range(TILE_R // LANES):
                                plsc.store_scatter(
                                    pbuf, [scat_base[m] + (c * PBLK + d)], vs[m])
                    riota = iov[pl.ds(0, LANES)]
                    ol = outls[b]
                    for c in range(2):
                        def comp(rg, carry, c=c):
                            pbase = riota + (rg * (8 * PAD_DIM) + c * PBLK)
                            obase = rg * (8 * EMBEDDING_DIM) + c * BLK
                            for u in range(8):
                                for d0 in range(0, EMBEDDING_DIM, LANES):
                                    v = plsc.load_gather(
                                        pbuf, [pbase + (u * PAD_DIM + d0)])
                                    ol[pl.ds(obase + (u * EMBEDDING_DIM + d0),
                                             LANES)] = v
                            return carry

                        lax.fori_loop(0, TILE_R // 8, comp, 0)
                    pltpu.async_copy(
                        ol, out_hbm.at[pl.ds((j0 + 2 * t) * BLK, 2 * BLK)],
                        ssem)
            return carry

        lax.fori_loop(0, n_outer, outer, 0)
        for p in range(NBUF - 1):
            pltpu.make_async_copy(
                outls[0], out_hbm.at[pl.ds(0, 2 * BLK)], ssem).wait()

        @pl.when(wid == 0)
        def _():
            pltpu.sync_copy(tail_hbm, tstag)
            pltpu.sync_copy(tstag, out_hbm.at[pl.ds(n_full * BLK, tail_words)])

    return sc_repack


def _make_sc_gather(n_flat):
    info = plsc.get_sparse_core_info()
    nc, ns = info.num_cores, info.num_subcores
    nw = nc * ns
    per_w = n_flat // nw
    assert per_w * nw == n_flat and per_w % (ROW_W * DMAS_PER_CHUNK) == 0
    n_idx_rows = per_w // ROW_W
    chunk = ROW_W * DMAS_PER_CHUNK
    n_chunks = per_w // chunk

    mesh = plsc.VectorSubcoreMesh(core_axis_name="c", subcore_axis_name="s")

    @functools.partial(
        pl.kernel,
        mesh=mesh,
        compiler_params=pltpu.CompilerParams(use_tc_tiling_on_sc=False),
        out_type=jax.ShapeDtypeStruct((n_flat, EMBEDDING_DIM), jnp.float32),
        scratch_types=[
            pltpu.VMEM((n_idx_rows, ROW_W), jnp.int32),
            pltpu.VMEM((n_idx_rows, ROW_W), jnp.int32),
            pltpu.VMEM((chunk, EMBEDDING_DIM), jnp.float32),
            pltpu.SemaphoreType.DMA,
        ],
    )
    def sc_gather(idx_hbm, off_hbm, table_hbm, out_hbm, idx_v, off_v, rows_v,
                  gsem):
        wid = lax.axis_index("s") * nc + lax.axis_index("c")
        base_row = wid * n_idx_rows
        pltpu.sync_copy(idx_hbm.at[pl.ds(base_row, n_idx_rows)], idx_v)
        pltpu.sync_copy(off_hbm, off_v)

        def add_body(r, carry):
            for k in range(ROW_W // LANES):
                sl = pl.ds(k * LANES, LANES)
                idx_v[r, sl] = idx_v[r, sl] + off_v[r, sl]
            return carry

        lax.fori_loop(0, n_idx_rows, add_body, 0)

        def chunk_body(t, carry):
            copies = []
            for b in range(DMAS_PER_CHUNK):
                copies.append(pltpu.async_copy(
                    table_hbm.at[idx_v.at[t * DMAS_PER_CHUNK + b]],
                    rows_v.at[pl.ds(b * ROW_W, ROW_W)],
                    gsem,
                ))
            for c in copies:
                c.wait()
            pltpu.sync_copy(
                rows_v,
                out_hbm.at[pl.ds((base_row + t * DMAS_PER_CHUNK) * ROW_W, chunk)],
            )
            return carry

        lax.fori_loop(0, n_chunks, chunk_body, 0)

    return sc_gather


def kernel(categorical_inputs, embedding_weight, offsets):
    batch, n_fields = categorical_inputs.shape
    n_flat = batch * n_fields
    idx_flat = categorical_inputs.astype(jnp.int32).reshape(n_flat // ROW_W, ROW_W)
    info = plsc.get_sparse_core_info()
    per_w = n_flat // (info.num_cores * info.num_subcores)
    off_pattern = jnp.tile(
        offsets[:n_fields].astype(jnp.int32), per_w // n_fields
    ).reshape(per_w // ROW_W, ROW_W)

    n_rows = embedding_weight.shape[0]
    n_full = n_rows // TILE_R
    tail = jax.lax.slice(
        embedding_weight, (n_full * TILE_R, 0), (n_rows, EMBEDDING_DIM)
    ).reshape(-1)
    iota16 = jnp.arange(LANES, dtype=jnp.int32)
    sc_repack = _make_sc_repack(n_rows)
    scratch = sc_repack(embedding_weight.T, tail, iota16)
    table_lin = scratch.reshape(-1, EMBEDDING_DIM)

    sc_gather = _make_sc_gather(n_flat)
    out_flat = sc_gather(idx_flat, off_pattern, table_lin)
    return out_flat.reshape(batch, n_fields, EMBEDDING_DIM)
